# baseline (speedup 1.0000x reference)
import functools

import jax
import jax.numpy as jnp
from jax import lax
from jax.experimental import pallas as pl
from jax.experimental.pallas import tpu as pltpu
from jax.experimental.pallas import tpu_sc as plsc

N = 10000
E = 320000
D = 128

NC = 2
NS = 16
NW = NC * NS

LPR = 128
EROWS = E // LPR
ROWS_PER_TILE = 80
EDGES_PER_TILE = ROWS_PER_TILE * LPR
E_PAD = NW * EDGES_PER_TILE
N_PAD = 10240
ACC_ROWS = 10240

KB = 16


def _sc_mesh():
    return plsc.VectorSubcoreMesh(core_axis_name="c", subcore_axis_name="s")


def _sc_degree(ei4_hbm, degp_out, src2_out, dst2_out,
               deg_sh, src_sh, dst_sh, spc, dpc, sblk, dblk,
               ones_v, zb_v, sem_d):
    i32 = jnp.int32
    c = lax.axis_index("c")
    s = lax.axis_index("s")
    wid = c * i32(NS) + s

    def _z(i, _):
        zb_v[pl.ds(i * i32(16), 16)] = jnp.zeros((16,), jnp.float32)
        return _
    lax.fori_loop(i32(0), i32((N_PAD // NS) // 16), _z, i32(0))
    pltpu.sync_copy(zb_v, deg_sh.at[pl.ds(s * i32(N_PAD // NS), N_PAD // NS)])

    def _o(i, _):
        ones_v[pl.ds(i * i32(16), 16)] = jnp.ones((16,), jnp.float32)
        return _
    lax.fori_loop(i32(0), i32(LPR // 16), _o, i32(0))

    plsc.subcore_barrier()

    lanes0 = lax.iota(jnp.int32, 16)
    g0 = wid * i32(ROWS_PER_TILE)

    for k in range(ROWS_PER_TILE // KB):
        g0k = g0 + i32(k * KB)
        gcl = jnp.minimum(g0k, i32(EROWS - KB))
        pltpu.sync_copy(
            ei4_hbm.at[i32(0), pl.ds(gcl * i32(2 * LPR), KB * 2 * LPR)], spc)
        pltpu.sync_copy(
            ei4_hbm.at[i32(1), pl.ds(gcl * i32(2 * LPR), KB * 2 * LPR)], dpc)

        def _row(r, _):
            g = g0k + r
            rbase = (g - gcl) * i32(2 * LPR)

            @pl.when(g < i32(EROWS))
            def _real():
                for t in range(LPR // 16):
                    ln = rbase + (lanes0 + i32(16 * t)) * i32(2)
                    sblk[r, pl.ds(i32(16 * t), 16)] = plsc.load_gather(
                        spc, [ln])
                    dblk[r, pl.ds(i32(16 * t), 16)] = plsc.load_gather(
                        dpc, [ln])

            @pl.when(g >= i32(EROWS))
            def _pad():
                for t in range(LPR // 16):
                    ln = lanes0 + i32(16 * t)
                    sblk[r, pl.ds(i32(16 * t), 16)] = ln
                    dblk[r, pl.ds(i32(16 * t), 16)] = (
                        i32(N) + (g * i32(LPR) + ln) % i32(ACC_ROWS - N))
            return _
        lax.fori_loop(i32(0), i32(KB), _row, i32(0))

        lrow = s * i32(ROWS_PER_TILE) + i32(k * KB)
        pltpu.sync_copy(sblk, src_sh.at[pl.ds(lrow, KB)])
        pltpu.sync_copy(dblk, dst_sh.at[pl.ds(lrow, KB)])

        def _fire(j, _):
            pltpu.async_copy(ones_v, deg_sh.at[dblk.at[j]], sem_d, add=True)
            return _
        lax.fori_loop(i32(0), i32(KB), _fire, i32(0))

        def _drain(j, _):
            pltpu.make_async_copy(ones_v, deg_sh.at[dblk.at[j]],
                                  sem_d).wait()
            return _
        lax.fori_loop(i32(0), i32(KB), _drain, i32(0))

    plsc.subcore_barrier()

    @pl.when(s == 0)
    def _():
        pltpu.sync_copy(deg_sh, degp_out.at[c])
    wr = (E_PAD // LPR // NC) // NS
    gbase = c * i32(E_PAD // LPR // NC) + s * i32(wr)
    pltpu.sync_copy(src_sh.at[pl.ds(s * i32(wr), wr)],
                    src2_out.at[pl.ds(gbase, wr)])
    pltpu.sync_copy(dst_sh.at[pl.ds(s * i32(wr), wr)],
                    dst2_out.at[pl.ds(gbase, wr)])


def _degree_partials(ei4):
    kern = pl.kernel(
        _sc_degree,
        out_type=[
            jax.ShapeDtypeStruct((NC, N_PAD), jnp.float32),
            jax.ShapeDtypeStruct((E_PAD // LPR, LPR), jnp.int32),
            jax.ShapeDtypeStruct((E_PAD // LPR, LPR), jnp.int32),
        ],
        mesh=_sc_mesh(),
        name="sc_degree",
        scratch_types=[
            pltpu.VMEM_SHARED((N_PAD,), jnp.float32),
            pltpu.VMEM_SHARED((E_PAD // LPR // NC, LPR), jnp.int32),
            pltpu.VMEM_SHARED((E_PAD // LPR // NC, LPR), jnp.int32),
            pltpu.VMEM((KB * 2 * LPR,), jnp.int32),
            pltpu.VMEM((KB * 2 * LPR,), jnp.int32),
            pltpu.VMEM((KB, LPR), jnp.int32),
            pltpu.VMEM((KB, LPR), jnp.int32),
            pltpu.VMEM((LPR,), jnp.float32),
            pltpu.VMEM((N_PAD // NS,), jnp.float32),
            pltpu.SemaphoreType.DMA,
        ],
        compiler_params=pltpu.CompilerParams(needs_layout_passes=False),
    )
    return kern(ei4)


def _tc_transform(x_ref, w_ref, b_ref, degp_ref, g_ref):
    h = lax.dot_general(x_ref[...], w_ref[...], (((1,), (1,)), ((), ())),
                        preferred_element_type=jnp.float32)
    h = h + b_ref[...]
    deg = degp_ref[0] + degp_ref[1] + 1.0
    dinv = lax.rsqrt(deg)
    g_ref[...] = h * dinv


def _transform(X, W, b2, degp3):
    R = 1000
    grid = (N // R,)
    return pl.pallas_call(
        _tc_transform,
        name="tc_transform",
        grid=grid,
        in_specs=[
            pl.BlockSpec((R, D), lambda i: (i, jnp.int32(0))),
            pl.BlockSpec((D, D), lambda i: (jnp.int32(0), jnp.int32(0))),
            pl.BlockSpec((1, D), lambda i: (jnp.int32(0), jnp.int32(0))),
            pl.BlockSpec((NC, R, 1), lambda i: (jnp.int32(0), i, jnp.int32(0))),
        ],
        out_specs=pl.BlockSpec((R, D), lambda i: (i, jnp.int32(0))),
        out_shape=jax.ShapeDtypeStruct((N, D), jnp.float32),
    )(X, W, b2, degp3)


def _sc_aggregate(g_hbm, src2_hbm, dst2_hbm, out_hbm,
                  acc_sh, src_v, dst_v, rows0_v, rows1_v, sem0, sem1):
    i32 = jnp.int32
    c = lax.axis_index("c")
    s = lax.axis_index("s")
    wid = c * i32(NS) + s

    def _z(i, _):
        rows0_v[i // i32(D // 16), pl.ds((i % i32(D // 16)) * i32(16), 16)] = (
            jnp.zeros((16,), jnp.float32))
        return _
    lax.fori_loop(i32(0), i32(LPR * (D // 16)), _z, i32(0))
    zrows = ACC_ROWS // NS

    def _zc(k, _):
        pltpu.sync_copy(rows0_v,
                        acc_sh.at[pl.ds(s * i32(zrows) + k * i32(LPR), LPR)])
        return _
    lax.fori_loop(i32(0), i32(zrows // LPR), _zc, i32(0))

    plsc.subcore_barrier()

    rbase = wid * i32(ROWS_PER_TILE)

    def _chunk(cidx, _):
        rb = rbase + cidx * i32(KB)
        pltpu.sync_copy(src2_hbm.at[pl.ds(rb, KB)], src_v)
        pltpu.sync_copy(dst2_hbm.at[pl.ds(rb, KB)], dst_v)
        pltpu.async_copy(g_hbm.at[src_v.at[i32(0)]], rows0_v, sem0)

        def _pair(k, _):
            j0 = k * i32(2)
            pltpu.make_async_copy(g_hbm.at[src_v.at[j0]], rows0_v,
                                  sem0).wait()
            pltpu.async_copy(g_hbm.at[src_v.at[j0 + i32(1)]], rows1_v, sem1)
            pltpu.sync_copy(rows0_v, acc_sh.at[dst_v.at[j0]], add=True)
            pltpu.make_async_copy(g_hbm.at[src_v.at[j0 + i32(1)]],
                                  rows1_v, sem1).wait()

            @pl.when(k < i32(KB // 2 - 1))
            def _prefetch():
                pltpu.async_copy(g_hbm.at[src_v.at[j0 + i32(2)]], rows0_v,
                                 sem0)
            pltpu.sync_copy(rows1_v, acc_sh.at[dst_v.at[j0 + i32(1)]],
                            add=True)
            return _
        lax.fori_loop(i32(0), i32(KB // 2), _pair, i32(0))
        return _
    lax.fori_loop(i32(0), i32(ROWS_PER_TILE // KB), _chunk, i32(0))

    plsc.subcore_barrier()

    wrows = ACC_ROWS // NS
    pltpu.sync_copy(acc_sh.at[pl.ds(s * i32(wrows), wrows)],
                    out_hbm.at[c, pl.ds(s * i32(wrows), wrows)])


def _aggregate_partials(G, src2, dst2):
    kern = pl.kernel(
        _sc_aggregate,
        out_type=jax.ShapeDtypeStruct((NC, ACC_ROWS, D), jnp.float32),
        mesh=_sc_mesh(),
        name="sc_aggregate",
        scratch_types=[
            pltpu.VMEM_SHARED((ACC_ROWS, D), jnp.float32),
            pltpu.VMEM((KB, LPR), jnp.int32),
            pltpu.VMEM((KB, LPR), jnp.int32),
            pltpu.VMEM((LPR, D), jnp.float32),
            pltpu.VMEM((LPR, D), jnp.float32),
            pltpu.SemaphoreType.DMA,
            pltpu.SemaphoreType.DMA,
        ],
    )
    return kern(G, src2, dst2)


def _tc_finalize(p_ref, g_ref, degp_ref, o_ref):
    deg = degp_ref[0] + degp_ref[1] + 1.0
    dinv = lax.rsqrt(deg)
    acc = (p_ref[0] + p_ref[1] + g_ref[...]) * dinv
    o_ref[...] = jnp.maximum(acc, 0.0)


def _finalize(P, SL, degp3):
    R = 1000
    grid = (N // R,)
    return pl.pallas_call(
        _tc_finalize,
        name="tc_finalize",
        grid=grid,
        in_specs=[
            pl.BlockSpec((NC, R, D), lambda i: (jnp.int32(0), i, jnp.int32(0))),
            pl.BlockSpec((R, D), lambda i: (i, jnp.int32(0))),
            pl.BlockSpec((NC, R, 1), lambda i: (jnp.int32(0), i, jnp.int32(0))),
        ],
        out_specs=pl.BlockSpec((R, D), lambda i: (i, jnp.int32(0))),
        out_shape=jax.ShapeDtypeStruct((N, D), jnp.float32),
    )(P, SL, degp3)


def kernel(X, edge_index, W, b):
    X = X.astype(jnp.float32)
    W = W.astype(jnp.float32)
    b2 = b.astype(jnp.float32).reshape(1, D)

    ei4 = lax.bitcast_convert_type(
        edge_index.astype(jnp.int64).reshape(2, EROWS, LPR),
        jnp.int32).reshape(2, EROWS * 2 * LPR)

    degp, src2, dst2 = _degree_partials(ei4)
    degp3 = degp.reshape(NC, N_PAD, 1)
    G = _transform(X, W, b2, degp3)
    P = _aggregate_partials(G, src2, dst2)
    return _finalize(P, G, degp3)


# speedup vs baseline: 8.0000x; 8.0000x over previous
import functools

import jax
import jax.numpy as jnp
from jax import lax
from jax.experimental import pallas as pl
from jax.experimental.pallas import tpu as pltpu
from jax.experimental.pallas import tpu_sc as plsc

N = 10000
E = 320000
D = 128

NC = 2
NS = 16
NW = NC * NS

LPR = 128
ROWS_PER_TILE = 80
EDGES_PER_TILE = ROWS_PER_TILE * LPR
E_PAD = NW * EDGES_PER_TILE
N_PAD = 10240
ACC_ROWS = 10240

KB = 16


def _sc_mesh():
    return plsc.VectorSubcoreMesh(core_axis_name="c", subcore_axis_name="s")


def _sc_degree(dst2_hbm, out_hbm, deg_sh, idx_v, ones_v, zb_v, sem_d):
    i32 = jnp.int32
    c = lax.axis_index("c")
    s = lax.axis_index("s")
    wid = c * i32(NS) + s

    def _z(i, _):
        zb_v[pl.ds(i * i32(16), 16)] = jnp.zeros((16,), jnp.float32)
        return _
    lax.fori_loop(i32(0), i32((N_PAD // NS) // 16), _z, i32(0))
    pltpu.sync_copy(zb_v, deg_sh.at[pl.ds(s * i32(N_PAD // NS), N_PAD // NS)])

    def _o(i, _):
        ones_v[pl.ds(i * i32(16), 16)] = jnp.ones((16,), jnp.float32)
        return _
    lax.fori_loop(i32(0), i32(LPR // 16), _o, i32(0))

    plsc.subcore_barrier()

    rbase = wid * i32(ROWS_PER_TILE)

    def _chunk(k, _):
        pltpu.sync_copy(dst2_hbm.at[pl.ds(rbase + k * i32(KB), KB)], idx_v)

        def _row(j, _):
            pltpu.async_copy(ones_v, deg_sh.at[idx_v.at[j]], sem_d, add=True)
            return _
        lax.fori_loop(i32(0), i32(KB), _row, i32(0))

        def _drain(j, _):
            pltpu.make_async_copy(ones_v, deg_sh.at[idx_v.at[j]],
                                  sem_d).wait()
            return _
        lax.fori_loop(i32(0), i32(KB), _drain, i32(0))
        return _
    lax.fori_loop(i32(0), i32(ROWS_PER_TILE // KB), _chunk, i32(0))

    plsc.subcore_barrier()

    @pl.when(s == 0)
    def _():
        pltpu.sync_copy(deg_sh, out_hbm.at[c])


def _degree_partials(dst2):
    kern = pl.kernel(
        _sc_degree,
        out_type=jax.ShapeDtypeStruct((NC, N_PAD), jnp.float32),
        mesh=_sc_mesh(),
        name="sc_degree",
        scratch_types=[
            pltpu.VMEM_SHARED((N_PAD,), jnp.float32),
            pltpu.VMEM((KB, LPR), jnp.int32),
            pltpu.VMEM((LPR,), jnp.float32),
            pltpu.VMEM((N_PAD // NS,), jnp.float32),
            pltpu.SemaphoreType.DMA,
        ],
    )
    return kern(dst2)


def _tc_transform(x_ref, w_ref, b_ref, degp_ref, g_ref):
    h = lax.dot_general(x_ref[...], w_ref[...], (((1,), (1,)), ((), ())),
                        preferred_element_type=jnp.float32)
    h = h + b_ref[...]
    deg = degp_ref[0] + degp_ref[1] + 1.0
    dinv = lax.rsqrt(deg)
    g_ref[...] = h * dinv


def _transform(X, W, b2, degp3):
    R = 1000
    grid = (N // R,)
    return pl.pallas_call(
        _tc_transform,
        name="tc_transform",
        grid=grid,
        in_specs=[
            pl.BlockSpec((R, D), lambda i: (i, jnp.int32(0))),
            pl.BlockSpec((D, D), lambda i: (jnp.int32(0), jnp.int32(0))),
            pl.BlockSpec((1, D), lambda i: (jnp.int32(0), jnp.int32(0))),
            pl.BlockSpec((NC, R, 1), lambda i: (jnp.int32(0), i, jnp.int32(0))),
        ],
        out_specs=pl.BlockSpec((R, D), lambda i: (i, jnp.int32(0))),
        out_shape=jax.ShapeDtypeStruct((N, D), jnp.float32),
    )(X, W, b2, degp3)


def _sc_aggregate(g_hbm, src2_hbm, dst2_hbm, out_hbm,
                  acc_sh, src_v, dst_v, rows0_v, rows1_v, sem0, sem1):
    i32 = jnp.int32
    c = lax.axis_index("c")
    s = lax.axis_index("s")
    wid = c * i32(NS) + s

    def _z(i, _):
        rows0_v[i // i32(D // 16), pl.ds((i % i32(D // 16)) * i32(16), 16)] = (
            jnp.zeros((16,), jnp.float32))
        return _
    lax.fori_loop(i32(0), i32(LPR * (D // 16)), _z, i32(0))
    zrows = ACC_ROWS // NS

    def _zc(k, _):
        pltpu.sync_copy(rows0_v,
                        acc_sh.at[pl.ds(s * i32(zrows) + k * i32(LPR), LPR)])
        return _
    lax.fori_loop(i32(0), i32(zrows // LPR), _zc, i32(0))

    plsc.subcore_barrier()

    rbase = wid * i32(ROWS_PER_TILE)

    def _chunk(cidx, _):
        rb = rbase + cidx * i32(KB)
        pltpu.sync_copy(src2_hbm.at[pl.ds(rb, KB)], src_v)
        pltpu.sync_copy(dst2_hbm.at[pl.ds(rb, KB)], dst_v)
        pltpu.async_copy(g_hbm.at[src_v.at[i32(0)]], rows0_v, sem0)

        def _pair(k, _):
            j0 = k * i32(2)
            pltpu.make_async_copy(g_hbm.at[src_v.at[j0]], rows0_v,
                                  sem0).wait()
            pltpu.async_copy(g_hbm.at[src_v.at[j0 + i32(1)]], rows1_v, sem1)
            pltpu.sync_copy(rows0_v, acc_sh.at[dst_v.at[j0]], add=True)
            pltpu.make_async_copy(g_hbm.at[src_v.at[j0 + i32(1)]],
                                  rows1_v, sem1).wait()

            @pl.when(k < i32(KB // 2 - 1))
            def _prefetch():
                pltpu.async_copy(g_hbm.at[src_v.at[j0 + i32(2)]], rows0_v,
                                 sem0)
            pltpu.sync_copy(rows1_v, acc_sh.at[dst_v.at[j0 + i32(1)]],
                            add=True)
            return _
        lax.fori_loop(i32(0), i32(KB // 2), _pair, i32(0))
        return _
    lax.fori_loop(i32(0), i32(ROWS_PER_TILE // KB), _chunk, i32(0))

    plsc.subcore_barrier()

    wrows = ACC_ROWS // NS
    pltpu.sync_copy(acc_sh.at[pl.ds(s * i32(wrows), wrows)],
                    out_hbm.at[c, pl.ds(s * i32(wrows), wrows)])


def _aggregate_partials(G, src2, dst2):
    kern = pl.kernel(
        _sc_aggregate,
        out_type=jax.ShapeDtypeStruct((NC, ACC_ROWS, D), jnp.float32),
        mesh=_sc_mesh(),
        name="sc_aggregate",
        scratch_types=[
            pltpu.VMEM_SHARED((ACC_ROWS, D), jnp.float32),
            pltpu.VMEM((KB, LPR), jnp.int32),
            pltpu.VMEM((KB, LPR), jnp.int32),
            pltpu.VMEM((LPR, D), jnp.float32),
            pltpu.VMEM((LPR, D), jnp.float32),
            pltpu.SemaphoreType.DMA,
            pltpu.SemaphoreType.DMA,
        ],
    )
    return kern(G, src2, dst2)


def _tc_finalize(p_ref, g_ref, degp_ref, o_ref):
    deg = degp_ref[0] + degp_ref[1] + 1.0
    dinv = lax.rsqrt(deg)
    acc = (p_ref[0] + p_ref[1] + g_ref[...]) * dinv
    o_ref[...] = jnp.maximum(acc, 0.0)


def _finalize(P, SL, degp3):
    R = 1000
    grid = (N // R,)
    return pl.pallas_call(
        _tc_finalize,
        name="tc_finalize",
        grid=grid,
        in_specs=[
            pl.BlockSpec((NC, R, D), lambda i: (jnp.int32(0), i, jnp.int32(0))),
            pl.BlockSpec((R, D), lambda i: (i, jnp.int32(0))),
            pl.BlockSpec((NC, R, 1), lambda i: (jnp.int32(0), i, jnp.int32(0))),
        ],
        out_specs=pl.BlockSpec((R, D), lambda i: (i, jnp.int32(0))),
        out_shape=jax.ShapeDtypeStruct((N, D), jnp.float32),
    )(P, SL, degp3)


def kernel(X, edge_index, W, b):
    X = X.astype(jnp.float32)
    W = W.astype(jnp.float32)
    b2 = b.astype(jnp.float32).reshape(1, D)

    src = edge_index[0].astype(jnp.int32)
    dst = edge_index[1].astype(jnp.int32)
    pad = E_PAD - E
    iota = jnp.arange(pad, dtype=jnp.int32)
    src_p = jnp.concatenate([src, iota % N])
    dst_p = jnp.concatenate([dst, N + iota % (ACC_ROWS - N)])
    src2 = src_p.reshape(E_PAD // LPR, LPR)
    dst2 = dst_p.reshape(E_PAD // LPR, LPR)

    degp = _degree_partials(dst2)
    degp3 = degp.reshape(NC, N_PAD, 1)
    G = _transform(X, W, b2, degp3)
    P = _aggregate_partials(G, src2, dst2)
    return _finalize(P, G, degp3)


# speedup vs baseline: 8.0796x; 1.0100x over previous
import jax
import jax.numpy as jnp
from jax import lax
from jax.experimental import pallas as pl
from jax.experimental.pallas import tpu as pltpu
from jax.experimental.pallas import tpu_sc as plsc

N = 10000
E = 320000
D = 128

NC = 2
NS = 16
NW = NC * NS

LPR = 128
ROWS_PER_TILE = 80
EDGES_PER_TILE = ROWS_PER_TILE * LPR
E_PAD = NW * EDGES_PER_TILE
N_PAD = 10240
ACC_ROWS = 10240

KB = 16


def _sc_mesh():
    return plsc.VectorSubcoreMesh(core_axis_name="c", subcore_axis_name="s")


def _sc_degree(dst2_hbm, out_hbm, deg_sh, idx_v, ones_v, zb_v, sem_d):
    i32 = jnp.int32
    c = lax.axis_index("c")
    s = lax.axis_index("s")
    wid = c * i32(NS) + s

    def _z(i, _):
        zb_v[pl.ds(i * i32(16), 16)] = jnp.zeros((16,), jnp.float32)
        return _
    lax.fori_loop(i32(0), i32((N_PAD // NS) // 16), _z, i32(0))
    pltpu.sync_copy(zb_v, deg_sh.at[pl.ds(s * i32(N_PAD // NS), N_PAD // NS)])

    def _o(i, _):
        ones_v[pl.ds(i * i32(16), 16)] = jnp.ones((16,), jnp.float32)
        return _
    lax.fori_loop(i32(0), i32(LPR // 16), _o, i32(0))

    plsc.subcore_barrier()

    rbase = wid * i32(ROWS_PER_TILE)

    def _chunk(k, _):
        pltpu.sync_copy(dst2_hbm.at[pl.ds(rbase + k * i32(KB), KB)], idx_v)

        def _row(j, _):
            pltpu.async_copy(ones_v, deg_sh.at[idx_v.at[j]], sem_d, add=True)
            return _
        lax.fori_loop(i32(0), i32(KB), _row, i32(0))

        def _drain(j, _):
            pltpu.make_async_copy(ones_v, deg_sh.at[idx_v.at[j]],
                                  sem_d).wait()
            return _
        lax.fori_loop(i32(0), i32(KB), _drain, i32(0))
        return _
    lax.fori_loop(i32(0), i32(ROWS_PER_TILE // KB), _chunk, i32(0))

    plsc.subcore_barrier()

    @pl.when(s == 0)
    def _():
        pltpu.sync_copy(deg_sh, out_hbm.at[c])


def _degree_partials(dst2):
    kern = pl.kernel(
        _sc_degree,
        out_type=jax.ShapeDtypeStruct((NC, N_PAD), jnp.float32),
        mesh=_sc_mesh(),
        name="sc_degree",
        scratch_types=[
            pltpu.VMEM_SHARED((N_PAD,), jnp.float32),
            pltpu.VMEM((KB, LPR), jnp.int32),
            pltpu.VMEM((LPR,), jnp.float32),
            pltpu.VMEM((N_PAD // NS,), jnp.float32),
            pltpu.SemaphoreType.DMA,
        ],
    )
    return kern(dst2)


def _tc_transform(x_ref, w_ref, b_ref, degp_ref, g_ref):
    h = lax.dot_general(x_ref[...], w_ref[...], (((1,), (1,)), ((), ())),
                        preferred_element_type=jnp.float32)
    h = h + b_ref[...]
    deg = degp_ref[0] + degp_ref[1] + 1.0
    dinv = lax.rsqrt(deg)
    g_ref[...] = h * dinv


def _transform(X, W, b2, degp3):
    R = 2000
    grid = (N // R,)
    return pl.pallas_call(
        _tc_transform,
        name="tc_transform",
        grid=grid,
        in_specs=[
            pl.BlockSpec((R, D), lambda i: (i, jnp.int32(0))),
            pl.BlockSpec((D, D), lambda i: (jnp.int32(0), jnp.int32(0))),
            pl.BlockSpec((1, D), lambda i: (jnp.int32(0), jnp.int32(0))),
            pl.BlockSpec((NC, R, 1), lambda i: (jnp.int32(0), i, jnp.int32(0))),
        ],
        out_specs=pl.BlockSpec((R, D), lambda i: (i, jnp.int32(0))),
        out_shape=jax.ShapeDtypeStruct((N, D), jnp.float32),
    )(X, W, b2, degp3)


def _sc_aggregate(g_hbm, src2_hbm, dst2_hbm, out_hbm,
                  acc_sh, src_v, dst_v, rows0_v, rows1_v, sem0, sem1):
    i32 = jnp.int32
    c = lax.axis_index("c")
    s = lax.axis_index("s")
    wid = c * i32(NS) + s

    def _z(i, _):
        rows0_v[i // i32(D // 16), pl.ds((i % i32(D // 16)) * i32(16), 16)] = (
            jnp.zeros((16,), jnp.float32))
        return _
    lax.fori_loop(i32(0), i32(LPR * (D // 16)), _z, i32(0))
    zrows = ACC_ROWS // NS

    def _zc(k, _):
        pltpu.sync_copy(rows0_v,
                        acc_sh.at[pl.ds(s * i32(zrows) + k * i32(LPR), LPR)])
        return _
    lax.fori_loop(i32(0), i32(zrows // LPR), _zc, i32(0))

    plsc.subcore_barrier()

    rbase = wid * i32(ROWS_PER_TILE)

    def _chunk(cidx, _):
        rb = rbase + cidx * i32(KB)
        pltpu.sync_copy(src2_hbm.at[pl.ds(rb, KB)], src_v)
        pltpu.sync_copy(dst2_hbm.at[pl.ds(rb, KB)], dst_v)
        pltpu.async_copy(g_hbm.at[src_v.at[i32(0)]], rows0_v, sem0)

        def _pair(k, _):
            j0 = k * i32(2)
            pltpu.make_async_copy(g_hbm.at[src_v.at[j0]], rows0_v,
                                  sem0).wait()
            pltpu.async_copy(g_hbm.at[src_v.at[j0 + i32(1)]], rows1_v, sem1)
            pltpu.sync_copy(rows0_v, acc_sh.at[dst_v.at[j0]], add=True)
            pltpu.make_async_copy(g_hbm.at[src_v.at[j0 + i32(1)]],
                                  rows1_v, sem1).wait()

            @pl.when(k < i32(KB // 2 - 1))
            def _prefetch():
                pltpu.async_copy(g_hbm.at[src_v.at[j0 + i32(2)]], rows0_v,
                                 sem0)
            pltpu.sync_copy(rows1_v, acc_sh.at[dst_v.at[j0 + i32(1)]],
                            add=True)
            return _
        lax.fori_loop(i32(0), i32(KB // 2), _pair, i32(0))
        return _
    lax.fori_loop(i32(0), i32(ROWS_PER_TILE // KB), _chunk, i32(0))

    plsc.subcore_barrier()

    wrows = ACC_ROWS // NS
    pltpu.sync_copy(acc_sh.at[pl.ds(s * i32(wrows), wrows)],
                    out_hbm.at[c, pl.ds(s * i32(wrows), wrows)])


def _aggregate_partials(G, src2, dst2):
    kern = pl.kernel(
        _sc_aggregate,
        out_type=jax.ShapeDtypeStruct((NC, ACC_ROWS, D), jnp.float32),
        mesh=_sc_mesh(),
        name="sc_aggregate",
        scratch_types=[
            pltpu.VMEM_SHARED((ACC_ROWS, D), jnp.float32),
            pltpu.VMEM((KB, LPR), jnp.int32),
            pltpu.VMEM((KB, LPR), jnp.int32),
            pltpu.VMEM((LPR, D), jnp.float32),
            pltpu.VMEM((LPR, D), jnp.float32),
            pltpu.SemaphoreType.DMA,
            pltpu.SemaphoreType.DMA,
        ],
    )
    return kern(G, src2, dst2)


def _tc_finalize(p_ref, g_ref, degp_ref, o_ref):
    deg = degp_ref[0] + degp_ref[1] + 1.0
    dinv = lax.rsqrt(deg)
    acc = (p_ref[0] + p_ref[1] + g_ref[...]) * dinv
    o_ref[...] = jnp.maximum(acc, 0.0)


def _finalize(P, SL, degp3):
    R = 1000
    grid = (N // R,)
    return pl.pallas_call(
        _tc_finalize,
        name="tc_finalize",
        grid=grid,
        in_specs=[
            pl.BlockSpec((NC, R, D), lambda i: (jnp.int32(0), i, jnp.int32(0))),
            pl.BlockSpec((R, D), lambda i: (i, jnp.int32(0))),
            pl.BlockSpec((NC, R, 1), lambda i: (jnp.int32(0), i, jnp.int32(0))),
        ],
        out_specs=pl.BlockSpec((R, D), lambda i: (i, jnp.int32(0))),
        out_shape=jax.ShapeDtypeStruct((N, D), jnp.float32),
    )(P, SL, degp3)


def kernel(X, edge_index, W, b):
    X = X.astype(jnp.float32)
    W = W.astype(jnp.float32)
    b2 = b.astype(jnp.float32).reshape(1, D)

    src = edge_index[0].astype(jnp.int32)
    dst = edge_index[1].astype(jnp.int32)
    pad = E_PAD - E
    iota = jnp.arange(pad, dtype=jnp.int32)
    src_p = jnp.concatenate([src, iota % N])
    dst_p = jnp.concatenate([dst, N + iota % (ACC_ROWS - N)])
    src2 = src_p.reshape(E_PAD // LPR, LPR)
    dst2 = dst_p.reshape(E_PAD // LPR, LPR)

    degp = _degree_partials(dst2)
    degp3 = degp.reshape(NC, N_PAD, 1)
    G = _transform(X, W, b2, degp3)
    P = _aggregate_partials(G, src2, dst2)
    return _finalize(P, G, degp3)


# speedup vs baseline: 8.1195x; 1.0049x over previous
import jax
import jax.numpy as jnp
from jax import lax
from jax.experimental import pallas as pl
from jax.experimental.pallas import tpu as pltpu
from jax.experimental.pallas import tpu_sc as plsc

N = 10000
E = 320000
D = 128

NC = 2
NS = 16
NW = NC * NS

LPR = 128
ROWS_PER_TILE = 80
EDGES_PER_TILE = ROWS_PER_TILE * LPR
E_PAD = NW * EDGES_PER_TILE
N_PAD = 10240
ACC_ROWS = 10240

KB = 16


def _sc_mesh():
    return plsc.VectorSubcoreMesh(core_axis_name="c", subcore_axis_name="s")


def _sc_degree(dst2_hbm, out_hbm, deg_sh, idx_v, ones_v, zb_v, sem_d):
    i32 = jnp.int32
    c = lax.axis_index("c")
    s = lax.axis_index("s")
    wid = c * i32(NS) + s

    def _z(i, _):
        zb_v[pl.ds(i * i32(16), 16)] = jnp.zeros((16,), jnp.float32)
        return _
    lax.fori_loop(i32(0), i32((N_PAD // NS) // 16), _z, i32(0))
    pltpu.sync_copy(zb_v, deg_sh.at[pl.ds(s * i32(N_PAD // NS), N_PAD // NS)])

    def _o(i, _):
        ones_v[pl.ds(i * i32(16), 16)] = jnp.ones((16,), jnp.float32)
        return _
    lax.fori_loop(i32(0), i32(LPR // 16), _o, i32(0))

    plsc.subcore_barrier()

    rbase = wid * i32(ROWS_PER_TILE)

    def _chunk(k, _):
        pltpu.sync_copy(dst2_hbm.at[pl.ds(rbase + k * i32(KB), KB)], idx_v)

        def _row(j, _):
            pltpu.async_copy(ones_v, deg_sh.at[idx_v.at[j]], sem_d, add=True)
            return _
        lax.fori_loop(i32(0), i32(KB), _row, i32(0))

        def _drain(j, _):
            pltpu.make_async_copy(ones_v, deg_sh.at[idx_v.at[j]],
                                  sem_d).wait()
            return _
        lax.fori_loop(i32(0), i32(KB), _drain, i32(0))
        return _
    lax.fori_loop(i32(0), i32(ROWS_PER_TILE // KB), _chunk, i32(0))

    plsc.subcore_barrier()

    @pl.when(s == 0)
    def _():
        pltpu.sync_copy(deg_sh, out_hbm.at[c])


def _degree_partials(dst2):
    kern = pl.kernel(
        _sc_degree,
        out_type=jax.ShapeDtypeStruct((NC, N_PAD), jnp.float32),
        mesh=_sc_mesh(),
        name="sc_degree",
        scratch_types=[
            pltpu.VMEM_SHARED((N_PAD,), jnp.float32),
            pltpu.VMEM((KB, LPR), jnp.int32),
            pltpu.VMEM((LPR,), jnp.float32),
            pltpu.VMEM((N_PAD // NS,), jnp.float32),
            pltpu.SemaphoreType.DMA,
        ],
    )
    return kern(dst2)


def _tc_transform(x_ref, w_ref, b_ref, degp_ref, g_ref):
    h = lax.dot_general(x_ref[...], w_ref[...], (((1,), (1,)), ((), ())),
                        preferred_element_type=jnp.float32)
    h = h + b_ref[...]
    deg = degp_ref[0] + degp_ref[1] + 1.0
    dinv = lax.rsqrt(deg)
    g_ref[...] = h * dinv


def _transform(X, W, b2, degp3):
    R = 2000
    grid = (N // R,)
    return pl.pallas_call(
        _tc_transform,
        name="tc_transform",
        grid=grid,
        in_specs=[
            pl.BlockSpec((R, D), lambda i: (i, jnp.int32(0))),
            pl.BlockSpec((D, D), lambda i: (jnp.int32(0), jnp.int32(0))),
            pl.BlockSpec((1, D), lambda i: (jnp.int32(0), jnp.int32(0))),
            pl.BlockSpec((NC, R, 1), lambda i: (jnp.int32(0), i, jnp.int32(0))),
        ],
        out_specs=pl.BlockSpec((R, D), lambda i: (i, jnp.int32(0))),
        out_shape=jax.ShapeDtypeStruct((N, D), jnp.float32),
    )(X, W, b2, degp3)


def _sc_aggregate(g_hbm, src2_hbm, dst2_hbm, out_hbm,
                  acc_sh, src_v, dst_v, rows0_v, rows1_v, sem0, sem1):
    i32 = jnp.int32
    c = lax.axis_index("c")
    s = lax.axis_index("s")
    wid = c * i32(NS) + s

    def _z(i, _):
        rows0_v[i // i32(D // 16), pl.ds((i % i32(D // 16)) * i32(16), 16)] = (
            jnp.zeros((16,), jnp.float32))
        return _
    lax.fori_loop(i32(0), i32(LPR * (D // 16)), _z, i32(0))
    zrows = ACC_ROWS // NS

    def _zc(k, _):
        pltpu.sync_copy(rows0_v,
                        acc_sh.at[pl.ds(s * i32(zrows) + k * i32(LPR), LPR)])
        return _
    lax.fori_loop(i32(0), i32(zrows // LPR), _zc, i32(0))

    plsc.subcore_barrier()

    rbase = wid * i32(ROWS_PER_TILE)

    def _chunk(cidx, _):
        rb = rbase + cidx * i32(KB)
        pltpu.sync_copy(src2_hbm.at[pl.ds(rb, KB)], src_v)
        pltpu.sync_copy(dst2_hbm.at[pl.ds(rb, KB)], dst_v)
        pltpu.async_copy(g_hbm.at[src_v.at[i32(0)]], rows0_v, sem0)

        def _pair(k, _):
            j0 = k * i32(2)
            pltpu.make_async_copy(g_hbm.at[src_v.at[j0]], rows0_v,
                                  sem0).wait()
            pltpu.async_copy(g_hbm.at[src_v.at[j0 + i32(1)]], rows1_v, sem1)
            pltpu.sync_copy(rows0_v, acc_sh.at[dst_v.at[j0]], add=True)
            pltpu.make_async_copy(g_hbm.at[src_v.at[j0 + i32(1)]],
                                  rows1_v, sem1).wait()

            @pl.when(k < i32(KB // 2 - 1))
            def _prefetch():
                pltpu.async_copy(g_hbm.at[src_v.at[j0 + i32(2)]], rows0_v,
                                 sem0)
            pltpu.sync_copy(rows1_v, acc_sh.at[dst_v.at[j0 + i32(1)]],
                            add=True)
            return _
        lax.fori_loop(i32(0), i32(KB // 2), _pair, i32(0))
        return _
    lax.fori_loop(i32(0), i32(ROWS_PER_TILE // KB), _chunk, i32(0))

    plsc.subcore_barrier()

    wrows = ACC_ROWS // NS
    pltpu.sync_copy(acc_sh.at[pl.ds(s * i32(wrows), wrows)],
                    out_hbm.at[c, pl.ds(s * i32(wrows), wrows)])


def _aggregate_partials(G, src2, dst2):
    kern = pl.kernel(
        _sc_aggregate,
        out_type=jax.ShapeDtypeStruct((NC, ACC_ROWS, D), jnp.float32),
        mesh=_sc_mesh(),
        name="sc_aggregate",
        scratch_types=[
            pltpu.VMEM_SHARED((ACC_ROWS, D), jnp.float32),
            pltpu.VMEM((KB, LPR), jnp.int32),
            pltpu.VMEM((KB, LPR), jnp.int32),
            pltpu.VMEM((LPR, D), jnp.float32),
            pltpu.VMEM((LPR, D), jnp.float32),
            pltpu.SemaphoreType.DMA,
            pltpu.SemaphoreType.DMA,
        ],
    )
    return kern(G, src2, dst2)


def _tc_finalize(p_ref, g_ref, degp_ref, o_ref):
    deg = degp_ref[0] + degp_ref[1] + 1.0
    dinv = lax.rsqrt(deg)
    acc = (p_ref[0] + p_ref[1] + g_ref[...]) * dinv
    o_ref[...] = jnp.maximum(acc, 0.0)


def _finalize(P, SL, degp3):
    R = 2000
    grid = (N // R,)
    return pl.pallas_call(
        _tc_finalize,
        name="tc_finalize",
        grid=grid,
        in_specs=[
            pl.BlockSpec((NC, R, D), lambda i: (jnp.int32(0), i, jnp.int32(0))),
            pl.BlockSpec((R, D), lambda i: (i, jnp.int32(0))),
            pl.BlockSpec((NC, R, 1), lambda i: (jnp.int32(0), i, jnp.int32(0))),
        ],
        out_specs=pl.BlockSpec((R, D), lambda i: (i, jnp.int32(0))),
        out_shape=jax.ShapeDtypeStruct((N, D), jnp.float32),
    )(P, SL, degp3)


def kernel(X, edge_index, W, b):
    X = X.astype(jnp.float32)
    W = W.astype(jnp.float32)
    b2 = b.astype(jnp.float32).reshape(1, D)

    src = edge_index[0].astype(jnp.int32)
    dst = edge_index[1].astype(jnp.int32)
    pad = E_PAD - E
    iota = jnp.arange(pad, dtype=jnp.int32)
    src_p = jnp.concatenate([src, iota % N])
    dst_p = jnp.concatenate([dst, N + iota % (ACC_ROWS - N)])
    src2 = src_p.reshape(E_PAD // LPR, LPR)
    dst2 = dst_p.reshape(E_PAD // LPR, LPR)

    degp = _degree_partials(dst2)
    degp3 = degp.reshape(NC, N_PAD, 1)
    G = _transform(X, W, b2, degp3)
    P = _aggregate_partials(G, src2, dst2)
    return _finalize(P, G, degp3)
